# baseline (speedup 1.0000x reference)
import functools

import jax
import jax.numpy as jnp
from jax.experimental import pallas as pl
from jax.experimental.pallas import tpu as pltpu

EPS = 1e-5


def _round_up(x, m):
    return (x + m - 1) // m * m


def _conv_stats_kernel(x_ref, w_ref, y_ref, stats_ref, *, h_out, w_out, wp,
                       kh_size, kw_size, chunk):
    h, w = h_out, w_out
    c = x_ref.shape[3]
    rows = h * wp
    img = x_ref[0].astype(jnp.bfloat16)
    zl = jnp.zeros((h, 1, c), jnp.bfloat16)
    zr = jnp.zeros((h, wp - w - 1, c), jnp.bfloat16)
    imgw = jnp.concatenate([zl, img, zr], axis=1).reshape(rows, c)
    ztop = jnp.zeros((wp, c), jnp.bfloat16)
    zbot = jnp.zeros((wp + 8, c), jnp.bfloat16)
    flat = jnp.concatenate([ztop, imgw, zbot], axis=0)

    mask = jax.lax.broadcasted_iota(jnp.int32, (chunk, 1), 0) % wp < w
    ones_c = jnp.ones((8, chunk), jnp.bfloat16)
    dn = (((1,), (0,)), ((), ()))
    psum = None
    psq = None
    for ch in range(rows // chunk):
        base = ch * chunk
        acc = None
        for kh in range(kh_size):
            for kw in range(kw_size):
                s = kh * wp + kw + base
                part = jax.lax.dot_general(
                    flat[s:s + chunk], w_ref[kh * kw_size + kw],
                    dimension_numbers=dn,
                    preferred_element_type=jnp.float32)
                acc = part if acc is None else acc + part
        yb = acc.astype(jnp.bfloat16)
        y_ref[0, base:base + chunk] = yb
        yv = jnp.where(mask, yb, jnp.bfloat16(0))
        ps = jax.lax.dot_general(ones_c, yv, dn,
                                 preferred_element_type=jnp.float32)
        pq = jax.lax.dot_general(ones_c, yv * yv, dn,
                                 preferred_element_type=jnp.float32)
        psum = ps if psum is None else psum + ps
        psq = pq if psq is None else psq + pq
    stats_ref[0] = jnp.concatenate([psum[0:1], psq[0:1]], axis=0)


def _bn_relu_kernel(y_ref, scale_ref, shift_ref, o_ref, *, h_out, w_out, wp):
    z = jnp.maximum(
        y_ref[0].astype(jnp.float32) * scale_ref[...] + shift_ref[...], 0.0)
    o_ref[0] = z.reshape(h_out, wp, z.shape[1])[:, :w_out, :]


def kernel(x_nchw, w_hwio, bias, gamma, beta):
    del bias
    N, Cin, H, W = x_nchw.shape
    KH, KW, _, Cout = w_hwio.shape
    WP = _round_up(W + KW - 1, 8)
    ROWS = H * WP
    CHUNK = 512 if ROWS % 512 == 0 else ROWS

    x_nhwc = jnp.transpose(x_nchw, (0, 2, 3, 1))
    w_packed = w_hwio.reshape(KH * KW, Cin, Cout).astype(jnp.bfloat16)

    cparams = pltpu.CompilerParams(
        dimension_semantics=("parallel",),
        vmem_limit_bytes=64 * 1024 * 1024)

    conv_flops = 2 * N * ROWS * KH * KW * Cin * Cout
    y, stats = pl.pallas_call(
        functools.partial(_conv_stats_kernel, h_out=H, w_out=W, wp=WP,
                          kh_size=KH, kw_size=KW, chunk=CHUNK),
        grid=(N,),
        in_specs=[
            pl.BlockSpec((1, H, W, Cin), lambda n: (n, 0, 0, 0)),
            pl.BlockSpec((KH * KW, Cin, Cout), lambda n: (0, 0, 0)),
        ],
        out_specs=[
            pl.BlockSpec((1, ROWS, Cout), lambda n: (n, 0, 0)),
            pl.BlockSpec((1, 2, Cout), lambda n: (n, 0, 0)),
        ],
        out_shape=[
            jax.ShapeDtypeStruct((N, ROWS, Cout), jnp.bfloat16),
            jax.ShapeDtypeStruct((N, 2, Cout), jnp.float32),
        ],
        compiler_params=cparams,
        cost_estimate=pl.CostEstimate(
            flops=int(conv_flops + 4 * N * ROWS * Cout),
            transcendentals=0,
            bytes_accessed=int(4 * x_nhwc.size + 2 * w_packed.size
                               + 2 * N * ROWS * Cout + 4 * N * 2 * Cout)),
    )(x_nhwc, w_packed)

    count = float(N * H * W)
    total = jnp.sum(stats, axis=0)
    mean = total[0] / count
    var = total[1] / count - mean * mean
    inv_std = jax.lax.rsqrt(var + EPS)
    scale = (gamma.astype(jnp.float32) * inv_std).reshape(1, Cout)
    shift = (beta.astype(jnp.float32) - mean * scale[0]).reshape(1, Cout)

    out = pl.pallas_call(
        functools.partial(_bn_relu_kernel, h_out=H, w_out=W, wp=WP),
        grid=(N,),
        in_specs=[
            pl.BlockSpec((1, ROWS, Cout), lambda n: (n, 0, 0)),
            pl.BlockSpec((1, Cout), lambda n: (0, 0)),
            pl.BlockSpec((1, Cout), lambda n: (0, 0)),
        ],
        out_specs=pl.BlockSpec((1, H, W, Cout), lambda n: (n, 0, 0, 0)),
        out_shape=jax.ShapeDtypeStruct((N, H, W, Cout), jnp.float32),
        compiler_params=cparams,
        cost_estimate=pl.CostEstimate(
            flops=int(3 * N * ROWS * Cout),
            transcendentals=0,
            bytes_accessed=int(2 * N * ROWS * Cout + 4 * N * H * W * Cout
                               + 8 * Cout)),
    )(y, scale, shift)

    return jnp.transpose(out, (0, 3, 1, 2))


# speedup vs baseline: 1.5055x; 1.5055x over previous
import functools

import jax
import jax.numpy as jnp
from jax.experimental import pallas as pl
from jax.experimental.pallas import tpu as pltpu

EPS = 1e-5


def _round_up(x, m):
    return (x + m - 1) // m * m


def _conv_stats_kernel(x_ref, w_ref, y_ref, stats_ref, *, h_out, w_out,
                       kh_size, kw_size):
    h, w = h_out, w_out
    c = x_ref.shape[3]
    rows = h * w
    img = x_ref[0].astype(jnp.bfloat16)
    zw = jnp.zeros((h, 1, c), jnp.bfloat16)
    imgw = jnp.concatenate([zw, img, zw], axis=1)
    zh = jnp.zeros((1, w + 2, c), jnp.bfloat16)
    slab = jnp.concatenate([zh, imgw, zh], axis=0)
    acc = None
    for kh in range(kh_size):
        row_slab = slab[kh:kh + h]
        for kw in range(kw_size):
            win = row_slab[:, kw:kw + w, :].reshape(rows, c)
            part = jax.lax.dot_general(
                win, w_ref[kh * kw_size + kw],
                dimension_numbers=(((1,), (0,)), ((), ())),
                preferred_element_type=jnp.float32)
            acc = part if acc is None else acc + part
    yb = acc.astype(jnp.bfloat16)
    y_ref[0] = yb
    ones_r = jnp.ones((8, rows), jnp.bfloat16)
    dn = (((1,), (0,)), ((), ()))
    psum = jax.lax.dot_general(ones_r, yb, dn,
                               preferred_element_type=jnp.float32)
    psq = jax.lax.dot_general(ones_r, yb * yb, dn,
                              preferred_element_type=jnp.float32)
    stats_ref[0] = jnp.concatenate([psum[0:1], psq[0:1]], axis=0)


def _bn_relu_kernel(y_ref, scale_ref, shift_ref, o_ref, *, h_out, w_out):
    z = jnp.maximum(
        y_ref[0].astype(jnp.float32) * scale_ref[...] + shift_ref[...], 0.0)
    o_ref[0] = z.reshape(h_out, w_out, z.shape[1])


def kernel(x_nchw, w_hwio, bias, gamma, beta):
    del bias
    N, Cin, H, W = x_nchw.shape
    KH, KW, _, Cout = w_hwio.shape
    HW = H * W

    x_nhwc = jnp.transpose(x_nchw, (0, 2, 3, 1))
    w_packed = w_hwio.reshape(KH * KW, Cin, Cout).astype(jnp.bfloat16)

    cparams = pltpu.CompilerParams(
        dimension_semantics=("parallel",),
        vmem_limit_bytes=64 * 1024 * 1024)

    conv_flops = 2 * N * HW * KH * KW * Cin * Cout
    y, stats = pl.pallas_call(
        functools.partial(_conv_stats_kernel, h_out=H, w_out=W, kh_size=KH,
                          kw_size=KW),
        grid=(N,),
        in_specs=[
            pl.BlockSpec((1, H, W, Cin), lambda n: (n, 0, 0, 0)),
            pl.BlockSpec((KH * KW, Cin, Cout), lambda n: (0, 0, 0)),
        ],
        out_specs=[
            pl.BlockSpec((1, HW, Cout), lambda n: (n, 0, 0)),
            pl.BlockSpec((1, 2, Cout), lambda n: (n, 0, 0)),
        ],
        out_shape=[
            jax.ShapeDtypeStruct((N, HW, Cout), jnp.bfloat16),
            jax.ShapeDtypeStruct((N, 2, Cout), jnp.float32),
        ],
        compiler_params=cparams,
        cost_estimate=pl.CostEstimate(
            flops=int(conv_flops + 4 * N * HW * Cout),
            transcendentals=0,
            bytes_accessed=int(4 * x_nhwc.size + 2 * w_packed.size
                               + 2 * N * HW * Cout + 4 * N * 2 * Cout)),
    )(x_nhwc, w_packed)

    count = float(N * HW)
    total = jnp.sum(stats, axis=0)
    mean = total[0] / count
    var = total[1] / count - mean * mean
    inv_std = jax.lax.rsqrt(var + EPS)
    scale = (gamma.astype(jnp.float32) * inv_std).reshape(1, Cout)
    shift = (beta.astype(jnp.float32) - mean * scale[0]).reshape(1, Cout)

    out = pl.pallas_call(
        functools.partial(_bn_relu_kernel, h_out=H, w_out=W),
        grid=(N,),
        in_specs=[
            pl.BlockSpec((1, HW, Cout), lambda n: (n, 0, 0)),
            pl.BlockSpec((1, Cout), lambda n: (0, 0)),
            pl.BlockSpec((1, Cout), lambda n: (0, 0)),
        ],
        out_specs=pl.BlockSpec((1, H, W, Cout), lambda n: (n, 0, 0, 0)),
        out_shape=jax.ShapeDtypeStruct((N, H, W, Cout), jnp.float32),
        compiler_params=cparams,
        cost_estimate=pl.CostEstimate(
            flops=int(3 * N * HW * Cout),
            transcendentals=0,
            bytes_accessed=int(2 * N * HW * Cout + 4 * N * HW * Cout
                               + 8 * Cout)),
    )(y, scale, shift)

    return jnp.transpose(out, (0, 3, 1, 2))
